# Initial kernel scaffold; baseline (speedup 1.0000x reference)
import jax
import jax.numpy as jnp
from jax.experimental import pallas as pl


def kernel(edge_attr0, edge_attr1, edge_attr2, edge_index, edge_index2, num_nodes, batch):
    raise NotImplementedError("write your pallas kernel here")


# speedup vs baseline: 3.5963x; 3.5963x over previous
import functools

import jax
import jax.numpy as jnp
from jax import lax
from jax.experimental import pallas as pl
from jax.experimental.pallas import tpu as pltpu
from jax.experimental.pallas import tpu_sc as plsc

N_NODES = 10000
N_EDGES = 320000
D = 128
G = 64
CHUNK = 80
NC = 2
NS = 16
NW = NC * NS
EDGE_CHUNKS = N_EDGES // CHUNK
NODE_CHUNKS = N_NODES // CHUNK


def _sc_body(attr0, attr1, attr2, dst1, dst2, batch_hbm,
             partial_out, counts_out,
             batch_v, idx_v, gid_v, gid_n, rows_v, ones_v,
             zero_v, acc_sh, cnt_sh):
    cid = lax.axis_index("c")
    sid = lax.axis_index("s")
    wid = sid * NC + cid

    zf = jnp.zeros((16,), jnp.float32)
    of = jnp.ones((16,), jnp.float32)

    def _zrow(r, _):
        for j in range(D // 16):
            zero_v[r, pl.ds(j * 16, 16)] = zf
        return 0
    lax.fori_loop(0, G, _zrow, 0)

    def _orow(r, _):
        for j in range(D // 16):
            ones_v[r, pl.ds(j * 16, 16)] = of
        return 0
    lax.fori_loop(0, CHUNK, _orow, 0)

    @pl.when(sid == 0)
    def _():
        pltpu.sync_copy(zero_v, acc_sh)
        pltpu.sync_copy(zero_v, cnt_sh)

    pltpu.sync_copy(batch_hbm, batch_v)
    plsc.subcore_barrier()

    def _edge_pass(attr_hbm, dst_hbm):
        def body(i, _):
            base = (i * NW + wid) * CHUNK
            base = pl.multiple_of(base, 8)
            pltpu.sync_copy(dst_hbm.at[pl.ds(base, CHUNK)], idx_v)
            for t in range(CHUNK // 16):
                iv = idx_v[pl.ds(t * 16, 16)]
                gid_v[pl.ds(t * 16, 16)] = plsc.load_gather(batch_v, [iv])
            pltpu.sync_copy(attr_hbm.at[pl.ds(base, CHUNK)], rows_v)
            pltpu.sync_copy(rows_v, acc_sh.at[gid_v], add=True)
            return 0
        lax.fori_loop(0, EDGE_CHUNKS // NW, body, 0)

    _edge_pass(attr1, dst1)
    _edge_pass(attr2, dst2)

    def _node_body(i, _):
        ci = i * NW + wid
        @pl.when(ci < NODE_CHUNKS)
        def _():
            base = pl.multiple_of(ci * CHUNK, 8)
            pltpu.sync_copy(batch_hbm.at[pl.ds(base, CHUNK)], gid_n)
            pltpu.sync_copy(attr0.at[pl.ds(base, CHUNK)], rows_v)
            pltpu.sync_copy(rows_v, acc_sh.at[gid_n], add=True)
            pltpu.sync_copy(ones_v, cnt_sh.at[gid_n], add=True)
        return 0
    lax.fori_loop(0, (NODE_CHUNKS + NW - 1) // NW, _node_body, 0)

    plsc.subcore_barrier()

    @pl.when(sid == 0)
    def _():
        pltpu.sync_copy(acc_sh, partial_out.at[cid])
        pltpu.sync_copy(cnt_sh, counts_out.at[cid])


_sc_pool = functools.partial(
    pl.kernel,
    out_type=[
        jax.ShapeDtypeStruct((NC, G, D), jnp.float32),
        jax.ShapeDtypeStruct((NC, G, D), jnp.float32),
    ],
    mesh=plsc.VectorSubcoreMesh(core_axis_name="c", subcore_axis_name="s"),
    compiler_params=pltpu.CompilerParams(needs_layout_passes=False),
    scratch_types=[
        pltpu.VMEM((N_NODES,), jnp.int32),
        pltpu.VMEM((CHUNK,), jnp.int32),
        pltpu.VMEM((CHUNK,), jnp.int32),
        pltpu.VMEM((CHUNK,), jnp.int32),
        pltpu.VMEM((CHUNK, D), jnp.float32),
        pltpu.VMEM((CHUNK, D), jnp.float32),
        pltpu.VMEM((G, D), jnp.float32),

        pltpu.VMEM_SHARED((G, D), jnp.float32),
        pltpu.VMEM_SHARED((G, D), jnp.float32),
    ],
)(_sc_body)


def _combine_body(p_ref, c_ref, o_ref):
    s = p_ref[0] + p_ref[1]
    cnt = c_ref[0, :, 0:1] + c_ref[1, :, 0:1]
    o_ref[...] = s / jnp.maximum(cnt, 1.0)


def kernel(edge_attr0, edge_attr1, edge_attr2, edge_index, edge_index2,
           num_nodes, batch):
    dst1 = edge_index[1].astype(jnp.int32)
    dst2 = edge_index2[1].astype(jnp.int32)
    batch32 = batch.astype(jnp.int32)
    partial, counts = _sc_pool(edge_attr0, edge_attr1, edge_attr2,
                               dst1, dst2, batch32)
    out = pl.pallas_call(
        _combine_body,
        out_shape=jax.ShapeDtypeStruct((G, D), jnp.float32),
    )(partial, counts)
    return out


# speedup vs baseline: 5.7915x; 1.6104x over previous
import functools

import jax
import jax.numpy as jnp
from jax import lax
from jax.experimental import pallas as pl
from jax.experimental.pallas import tpu as pltpu
from jax.experimental.pallas import tpu_sc as plsc

N_NODES = 10000
N_EDGES = 320000
D = 128
G = 64
CHUNK = 80
NC = 2
NS = 16
NW = NC * NS
EDGE_CHUNKS = N_EDGES // CHUNK
NODE_CHUNKS = N_NODES // CHUNK
EPW = N_EDGES // NW
CPW = EPW // CHUNK


def _sc_body(attr0, attr1, attr2, dst1, dst2, batch_hbm,
             partial_out, counts_out,
             batch_v, idx_all, gid_e, gid_n, rows_v, rows_b, ones_v,
             zero_v, acc_sh, cnt_sh, sem0, sem1):
    cid = lax.axis_index("c")
    sid = lax.axis_index("s")
    wid = sid * NC + cid

    zf = jnp.zeros((16,), jnp.float32)
    of = jnp.ones((16,), jnp.float32)

    def _zrow(r, _):
        for j in range(D // 16):
            zero_v[r, pl.ds(j * 16, 16)] = zf
        return 0
    lax.fori_loop(0, G, _zrow, 0)

    def _orow(r, _):
        for j in range(D // 16):
            ones_v[r, pl.ds(j * 16, 16)] = of
        return 0
    lax.fori_loop(0, CHUNK, _orow, 0)

    @pl.when(sid == 0)
    def _():
        pltpu.sync_copy(zero_v, acc_sh)
        pltpu.sync_copy(zero_v, cnt_sh)

    pltpu.sync_copy(batch_hbm, batch_v)
    plsc.subcore_barrier()

    def _edge_pass(attr_hbm, dst_hbm):
        wbase = pl.multiple_of(wid * EPW, 8)
        pltpu.sync_copy(dst_hbm.at[pl.ds(wbase, EPW)], idx_all)

        def _g(j, _):
            for u in range(CHUNK // 16):
                iv = idx_all[pl.ds(j * CHUNK + u * 16, 16)]
                gid_e[j, pl.ds(u * 16, 16)] = plsc.load_gather(batch_v, [iv])
            return 0
        lax.fori_loop(0, CPW, _g, 0)

        def _fill(buf, sem, ci):
            base = pl.multiple_of(wbase + ci * CHUNK, 8)
            pltpu.async_copy(attr_hbm.at[pl.ds(base, CHUNK)], buf, sem)

        def _wait(buf, sem):
            pltpu.make_async_copy(attr_hbm.at[pl.ds(0, CHUNK)], buf, sem).wait()

        _fill(rows_v, sem0, 0)

        def body(k, _):
            i0 = k * 2
            _wait(rows_v, sem0)
            _fill(rows_b, sem1, i0 + 1)
            pltpu.sync_copy(rows_v, acc_sh.at[gid_e.at[i0]], add=True)
            _wait(rows_b, sem1)
            _fill(rows_v, sem0, i0 + 2)
            pltpu.sync_copy(rows_b, acc_sh.at[gid_e.at[i0 + 1]], add=True)
            return 0
        lax.fori_loop(0, CPW // 2, body, 0)
        _wait(rows_v, sem0)
        pltpu.sync_copy(rows_v, acc_sh.at[gid_e.at[CPW - 1]], add=True)

    _edge_pass(attr1, dst1)
    _edge_pass(attr2, dst2)

    def _node_body(i, _):
        ci = i * NW + wid
        @pl.when(ci < NODE_CHUNKS)
        def _():
            base = pl.multiple_of(ci * CHUNK, 8)
            pltpu.sync_copy(batch_hbm.at[pl.ds(base, CHUNK)], gid_n)
            pltpu.sync_copy(attr0.at[pl.ds(base, CHUNK)], rows_v)
            pltpu.sync_copy(rows_v, acc_sh.at[gid_n], add=True)
            pltpu.sync_copy(ones_v, cnt_sh.at[gid_n], add=True)
        return 0
    lax.fori_loop(0, (NODE_CHUNKS + NW - 1) // NW, _node_body, 0)

    plsc.subcore_barrier()

    @pl.when(sid == 0)
    def _():
        pltpu.sync_copy(acc_sh, partial_out.at[cid])
        pltpu.sync_copy(cnt_sh, counts_out.at[cid])


_sc_pool = functools.partial(
    pl.kernel,
    out_type=[
        jax.ShapeDtypeStruct((NC, G, D), jnp.float32),
        jax.ShapeDtypeStruct((NC, G, D), jnp.float32),
    ],
    mesh=plsc.VectorSubcoreMesh(core_axis_name="c", subcore_axis_name="s"),
    compiler_params=pltpu.CompilerParams(needs_layout_passes=False),
    scratch_types=[
        pltpu.VMEM((N_NODES,), jnp.int32),
        pltpu.VMEM((EPW,), jnp.int32),
        pltpu.VMEM((CPW, CHUNK), jnp.int32),
        pltpu.VMEM((CHUNK,), jnp.int32),
        pltpu.VMEM((CHUNK, D), jnp.float32),
        pltpu.VMEM((CHUNK, D), jnp.float32),
        pltpu.VMEM((CHUNK, D), jnp.float32),
        pltpu.VMEM((G, D), jnp.float32),
        pltpu.VMEM_SHARED((G, D), jnp.float32),
        pltpu.VMEM_SHARED((G, D), jnp.float32),
        pltpu.SemaphoreType.DMA,
        pltpu.SemaphoreType.DMA,
    ],
)(_sc_body)


def _combine_body(p_ref, c_ref, o_ref):
    s = p_ref[0] + p_ref[1]
    cnt = c_ref[0, :, 0:1] + c_ref[1, :, 0:1]
    o_ref[...] = s / jnp.maximum(cnt, 1.0)


def kernel(edge_attr0, edge_attr1, edge_attr2, edge_index, edge_index2,
           num_nodes, batch):
    dst1 = edge_index[1].astype(jnp.int32)
    dst2 = edge_index2[1].astype(jnp.int32)
    batch32 = batch.astype(jnp.int32)
    partial, counts = _sc_pool(edge_attr0, edge_attr1, edge_attr2,
                               dst1, dst2, batch32)
    out = pl.pallas_call(
        _combine_body,
        out_shape=jax.ShapeDtypeStruct((G, D), jnp.float32),
    )(partial, counts)
    return out
